# Initial kernel scaffold; baseline (speedup 1.0000x reference)
import jax
import jax.numpy as jnp
from jax.experimental import pallas as pl


def kernel(pos, x, edge_index, batch, W_b1, b_b1, W_b2, b_b2, W_f1, b_f1, W_f2, b_f2, W_emb, Wk0, Wf0, cb0, lns0, lnb0, W1_0, b1_0, W2_0, b2_0, Wr0, br0, Wk1, Wf1, cb1, lns1, lnb1, W1_1, b1_1, W2_1, b2_1, Wr1, br1):
    raise NotImplementedError("write your pallas kernel here")


# speedup vs baseline: 4.0867x; 4.0867x over previous
import functools
import math

import jax
import jax.numpy as jnp
import numpy as np
from jax.experimental import pallas as pl

N_NODES = 10000
E_EDGES = 160000
IN_DIM = 128
HID = 64
BASIS = 32
N_ORI = 8
OUT_DIM = 64
OUT_VEC = 16
BATCH_SIZE = 16
WIDEN = 4


def _fibonacci_lattice(n, offset=0.5):
    i = jnp.arange(n)
    theta = math.pi * i * (1 + math.sqrt(5)) % (2 * math.pi)
    phi = jnp.arccos(1 - 2 * (i + offset) / (n - 1 + 2 * offset))
    return jnp.stack((jnp.sin(phi) * jnp.cos(theta),
                      jnp.sin(phi) * jnp.sin(theta), jnp.cos(phi)), axis=-1)


def _fibonacci_lattice_np(n=N_ORI, offset=0.5):
    i = np.arange(n)
    theta = math.pi * i * (1 + math.sqrt(5)) % (2 * math.pi)
    phi = np.arccos(1 - 2 * (i + offset) / (n - 1 + 2 * offset))
    return np.stack((np.sin(phi) * np.cos(theta), np.sin(phi) * np.sin(theta),
                     np.cos(phi)), axis=-1).astype(np.float32)


def _poly_features(x, degree):
    feats = [x]
    for _ in range(1, degree + 1):
        nf = jnp.einsum('...i,...j->...ij', feats[-1], x) \
            .reshape(x.shape[:-1] + (-1,))
        feats.append(nf)
    return jnp.concatenate(feats, axis=-1)


def _emb_body(x_ref, w_ref, o_ref):
    o_ref[...] = jnp.dot(x_ref[...], w_ref[...],
                         preferred_element_type=jnp.float32)


def _emb(x, W_emb, tile=2000):
    n = x.shape[0]
    return pl.pallas_call(
        _emb_body,
        grid=(n // tile,),
        in_specs=[
            pl.BlockSpec((tile, IN_DIM), lambda i: (i, 0)),
            pl.BlockSpec((IN_DIM, HID), lambda i: (0, 0)),
        ],
        out_specs=pl.BlockSpec((tile, HID), lambda i: (i, 0)),
        out_shape=jax.ShapeDtypeStruct((n, HID), jnp.float32),
    )(x, W_emb)


def _msg_body(first_layer, tile, invs_ref, hs_ref, wb1_ref, bb1_ref, wb2_ref,
              bb2_ref, wk_ref, o_ref):
    bb1 = bb1_ref[0, :][None, :]
    bb2 = bb2_ref[0, :][None, :]
    msg_parts = []
    for o in range(N_ORI):
        u = invs_ref[:, o:o + 1]
        v = invs_ref[:, N_ORI + o:N_ORI + o + 1]
        uu = u * u
        uv = u * v
        vu = v * u
        vv = v * v
        f = jnp.concatenate(
            [u, v, uu, uv, vu, vv,
             uu * u, uu * v, uv * u, uv * v,
             vu * u, vu * v, vv * u, vv * v], axis=-1)
        hid = jax.nn.gelu(
            jnp.dot(f, wb1_ref[...], preferred_element_type=jnp.float32)
            + bb1)
        kb = jax.nn.gelu(
            jnp.dot(hid, wb2_ref[...], preferred_element_type=jnp.float32)
            + bb2)
        kbw = jnp.dot(kb, wk_ref[...], preferred_element_type=jnp.float32)
        if first_layer:
            msg_parts.append(kbw * hs_ref[...])
        else:
            msg_parts.append(kbw * hs_ref[:, o * HID:(o + 1) * HID])
    o_ref[...] = jnp.concatenate(msg_parts, axis=-1)


def _msg_kernel(invs, hsend, W_b1, b_b1, W_b2, b_b2, Wk, first_layer, tile):
    e = invs.shape[0]
    hs_cols = HID if first_layer else N_ORI * HID
    return pl.pallas_call(
        functools.partial(_msg_body, first_layer, tile),
        grid=(e // tile,),
        in_specs=[
            pl.BlockSpec((tile, 16), lambda i: (i, 0)),
            pl.BlockSpec((tile, hs_cols), lambda i: (i, 0)),
            pl.BlockSpec((14, HID), lambda i: (0, 0)),
            pl.BlockSpec((1, HID), lambda i: (0, 0)),
            pl.BlockSpec((HID, BASIS), lambda i: (0, 0)),
            pl.BlockSpec((1, BASIS), lambda i: (0, 0)),
            pl.BlockSpec((BASIS, HID), lambda i: (0, 0)),
        ],
        out_specs=pl.BlockSpec((tile, N_ORI * HID), lambda i: (i, 0)),
        out_shape=jax.ShapeDtypeStruct((e, N_ORI * HID), jnp.float32),
    )(invs, hsend, W_b1, b_b1, W_b2, b_b2, Wk)


def _node_body(first_layer, tile, x2_ref, hp_ref,
               lns_ref, lnb_ref, w1_ref, b1_ref, w2_ref, b2_ref, h_out_ref):
    lns = lns_ref[0, :][None, :]
    lnb = lnb_ref[0, :][None, :]
    b1 = b1_ref[0, :][None, :]
    b2 = b2_ref[0, :][None, :]
    h_parts = []
    for p in range(N_ORI):
        x2 = x2_ref[:, p * HID:(p + 1) * HID]
        mu = jnp.mean(x2, axis=-1, keepdims=True)
        var = jnp.mean((x2 - mu) ** 2, axis=-1, keepdims=True)
        xn = (x2 - mu) / jnp.sqrt(var + 1e-6) * lns + lnb
        y = jnp.dot(
            jax.nn.gelu(jnp.dot(xn, w1_ref[...],
                                preferred_element_type=jnp.float32) + b1),
            w2_ref[...], preferred_element_type=jnp.float32) + b2
        if first_layer:
            h = y + hp_ref[...]
        else:
            h = y + hp_ref[:, p * HID:(p + 1) * HID]
        h_parts.append(h)
    h_out_ref[...] = jnp.concatenate(h_parts, axis=-1)


def _node_kernel(x2, hprev, lns, lnb, W1, b1, W2, b2, first_layer,
                 tile=1000):
    n = x2.shape[0]
    hp_cols = HID if first_layer else N_ORI * HID
    return pl.pallas_call(
        functools.partial(_node_body, first_layer, tile),
        grid=(n // tile,),
        in_specs=[
            pl.BlockSpec((tile, N_ORI * HID), lambda i: (i, 0)),
            pl.BlockSpec((tile, hp_cols), lambda i: (i, 0)),
            pl.BlockSpec((1, HID), lambda i: (0, 0)),
            pl.BlockSpec((1, HID), lambda i: (0, 0)),
            pl.BlockSpec((HID, WIDEN * HID), lambda i: (0, 0)),
            pl.BlockSpec((1, WIDEN * HID), lambda i: (0, 0)),
            pl.BlockSpec((WIDEN * HID, HID), lambda i: (0, 0)),
            pl.BlockSpec((1, HID), lambda i: (0, 0)),
        ],
        out_specs=pl.BlockSpec((tile, N_ORI * HID), lambda i: (i, 0)),
        out_shape=jax.ShapeDtypeStruct((n, N_ORI * HID), jnp.float32),
    )(x2, hprev, lns, lnb, W1, b1, W2, b2)


def _final_body(tile, h1_ref, h2_ref, wr0s_ref, wr1s_ref,
                wr0v_ref, wr1v_ref, bs_ref, bv_ref, rs_ref, rv_ref):
    bs = bs_ref[0, :][None, :]
    bv = bv_ref[0, :][None, :]
    rs_parts = []
    rv_parts = []
    for p in range(N_ORI):
        h1p = h1_ref[:, p * HID:(p + 1) * HID]
        h2p = h2_ref[:, p * HID:(p + 1) * HID]
        rs_parts.append(0.5 * (
            jnp.dot(h1p, wr0s_ref[...], preferred_element_type=jnp.float32)
            + jnp.dot(h2p, wr1s_ref[...], preferred_element_type=jnp.float32)
            + bs))
        rv_parts.append(0.5 * (
            jnp.dot(h1p, wr0v_ref[...], preferred_element_type=jnp.float32)
            + jnp.dot(h2p, wr1v_ref[...], preferred_element_type=jnp.float32)
            + bv))
    rs_ref[...] = jnp.concatenate(rs_parts, axis=-1)
    rv_ref[...] = jnp.concatenate(rv_parts, axis=-1)


def _final_kernel(h1, h2, Wr0s, Wr1s, Wr0v, Wr1v, bs, bv, tile=2000):
    n = h1.shape[0]
    return pl.pallas_call(
        functools.partial(_final_body, tile),
        grid=(n // tile,),
        in_specs=[
            pl.BlockSpec((tile, N_ORI * HID), lambda i: (i, 0)),
            pl.BlockSpec((tile, N_ORI * HID), lambda i: (i, 0)),
            pl.BlockSpec((HID, OUT_DIM), lambda i: (0, 0)),
            pl.BlockSpec((HID, OUT_DIM), lambda i: (0, 0)),
            pl.BlockSpec((HID, OUT_VEC), lambda i: (0, 0)),
            pl.BlockSpec((HID, OUT_VEC), lambda i: (0, 0)),
            pl.BlockSpec((1, OUT_DIM), lambda i: (0, 0)),
            pl.BlockSpec((1, OUT_VEC), lambda i: (0, 0)),
        ],
        out_specs=[
            pl.BlockSpec((tile, N_ORI * OUT_DIM), lambda i: (i, 0)),
            pl.BlockSpec((tile, N_ORI * OUT_VEC), lambda i: (i, 0)),
        ],
        out_shape=[
            jax.ShapeDtypeStruct((n, N_ORI * OUT_DIM), jnp.float32),
            jax.ShapeDtypeStruct((n, N_ORI * OUT_VEC), jnp.float32),
        ],
    )(h1, h2, Wr0s, Wr1s, Wr0v, Wr1v, bs, bv)


def _gather_rows(table, idx):
    return jnp.take(table, idx, axis=0)


def _segment_sum(msg, dst):
    return jax.ops.segment_sum(msg, dst, num_segments=N_NODES)


def kernel(pos, x, edge_index, batch, W_b1, b_b1, W_b2, b_b2, W_f1, b_f1,
           W_f2, b_f2, W_emb, Wk0, Wf0, cb0, lns0, lnb0, W1_0, b1_0, W2_0,
           b2_0, Wr0, br0, Wk1, Wf1, cb1, lns1, lnb1, W1_1, b1_1, W2_1, b2_1,
           Wr1, br1):
    send = edge_index[0]
    dst = edge_index[1]
    ori = _fibonacci_lattice(N_ORI).astype(pos.dtype)
    ori_const = _fibonacci_lattice_np()

    rel = (pos[send] - pos[dst])[:, None, :]
    inv1 = (rel * ori[None, :, :]).sum(axis=-1, keepdims=True)
    inv2 = jnp.linalg.norm(rel - inv1 * ori[None, :, :], axis=-1,
                           keepdims=True)
    invs = jnp.concatenate([inv1[:, :, 0], inv2[:, :, 0]], axis=-1)

    inv3 = (ori[None, :, :] * ori[:, None, :]).sum(-1, keepdims=True)
    fb = jax.nn.gelu(jax.nn.gelu(
        _poly_features(inv3, 2) @ W_f1 + b_f1) @ W_f2 + b_f2)
    fk0 = fb @ Wf0
    fk1 = fb @ Wf1

    h0 = _emb(x, W_emb)

    hs0 = _gather_rows(h0, send)
    msg0 = _msg_kernel(invs, hs0, W_b1, b_b1[None, :], W_b2, b_b2[None, :],
                       Wk0, first_layer=True, tile=800)
    x1_0 = _segment_sum(msg0, dst)
    x2_0 = (jnp.einsum('boc,poc->bpc', x1_0.reshape(N_NODES, N_ORI, HID),
                       fk0) / N_ORI + cb0).reshape(N_NODES, N_ORI * HID)
    h1 = _node_kernel(x2_0, h0, lns0[None, :],
                      lnb0[None, :], W1_0, b1_0[None, :], W2_0,
                      b2_0[None, :], first_layer=True)

    hs1 = _gather_rows(h1, send)
    msg1 = _msg_kernel(invs, hs1, W_b1, b_b1[None, :], W_b2, b_b2[None, :],
                       Wk1, first_layer=False, tile=800)
    x1_1 = _segment_sum(msg1, dst)
    x2_1 = (jnp.einsum('boc,poc->bpc', x1_1.reshape(N_NODES, N_ORI, HID),
                       fk1) / N_ORI + cb1).reshape(N_NODES, N_ORI * HID)
    h2 = _node_kernel(x2_1, h1, lns1[None, :],
                      lnb1[None, :], W1_1, b1_1[None, :], W2_1,
                      b2_1[None, :], first_layer=False)

    rs, rv = _final_kernel(
        h1, h2,
        Wr0[:, :OUT_DIM], Wr1[:, :OUT_DIM],
        Wr0[:, OUT_DIM:], Wr1[:, OUT_DIM:],
        (br0[:OUT_DIM] + br1[:OUT_DIM])[None, :],
        (br0[OUT_DIM:] + br1[OUT_DIM:])[None, :])
    out_scalar = rs.reshape(N_NODES, N_ORI, OUT_DIM).mean(axis=-2)
    out_vector_nodes = jnp.einsum(
        'boc,od->bcd', rv.reshape(N_NODES, N_ORI, OUT_VEC), ori) / N_ORI
    out_s = jax.ops.segment_sum(out_scalar, batch, num_segments=BATCH_SIZE)
    out_vector = jax.ops.segment_sum(out_vector_nodes, batch,
                                     num_segments=BATCH_SIZE)
    return (out_s, out_vector)
